# baseline (speedup 1.0000x reference)
import jax
import jax.numpy as jnp
from jax import lax
from jax.experimental import pallas as pl
from jax.experimental.pallas import tpu as pltpu
from jax.experimental.pallas import tpu_sc as plsc

B = 4
V = 50000
F = 100000
FT = 128
NT = (F + FT - 1) // FT
F_PAD = NT * FT
TPW = 25
DEPTH = 3
START_MUL = 25006
NVT = 391
V_PAD = NVT * FT
VTPW = 25
VSTART_MUL = 25008


def _rsqrt16(x):
    i = plsc.bitcast(x, jnp.int32)
    i = jnp.int32(0x5F3759DF) - (i >> 1)
    y = plsc.bitcast(i, jnp.float32)
    xh = 0.5 * x
    for _ in range(2):
        y = y * (1.5 - xh * y * y)
    return y


def _body(verts_hbm, fidx_hbm, out_hbm, *refs):
    planes = refs[0:2]
    idx_v = refs[2]
    vb = refs[3:6]
    ob = refs[6:9]
    cb = refs[9:11]
    rb = refs[11:13]
    st = refs[13:16]
    so = refs[16:19]
    sv = refs[19:21]
    sb = refs[21:23]

    sid = lax.axis_index("s")
    wid = sid * 2 + lax.axis_index("c")
    start = (wid * START_MUL) >> 10
    lane = lax.iota(jnp.int32, 16)

    vstart = (sid * VSTART_MUL) >> 10

    def fire_in(vt, p):
        for c in range(3):
            for b in range(B):
                pltpu.async_copy(
                    verts_hbm.at[c, vt, b], cb[p].at[c * 4 + b], sv[p]
                )

    def wait_in(p):
        for _ in range(12):
            pltpu.make_async_copy(
                verts_hbm.at[0, 0, 0], cb[0].at[0], sv[p]
            ).wait()

    def permute(p):
        for b in range(B):
            for c in range(3):
                col = jnp.full((16,), 4 * (b & 1) + c, jnp.int32)
                for i in range(8):
                    v = cb[p][c * 4 + b, pl.ds(16 * i, 16)]
                    plsc.store_scatter(
                        rb[p], [jnp.full((16,), b >> 1, jnp.int32),
                                16 * i + lane, col], v
                    )

    def fire_bout(vt, p):
        for pp in range(2):
            pltpu.async_copy(
                rb[p].at[pp], planes[pp].at[pl.ds(FT * vt, FT)], sb[p]
            )

    def wait_bout(p):
        for pp in range(2):
            pltpu.make_async_copy(
                rb[0].at[0], planes[0].at[pl.ds(0, FT)], sb[p]
            ).wait()

    fire_in(vstart, 0)

    def bslot(t, p):
        fire_in(t + 1, 1 - p)
        wait_in(p)

        @pl.when(t >= vstart + 2)
        def _():
            wait_bout(p)

        permute(p)
        fire_bout(t, p)

    def bouter(o, _):
        bslot(vstart + 2 * o, 0)
        bslot(vstart + 2 * o + 1, 1)
        return ()

    lax.fori_loop(0, (VTPW - 1) // 2, bouter, ())
    wait_in(0)
    wait_bout(0)
    permute(0)
    fire_bout(vstart + VTPW - 1, 0)
    wait_bout(1)
    wait_bout(0)
    plsc.subcore_barrier()

    pltpu.sync_copy(fidx_hbm.at[pl.ds(start, TPW)], idx_v)

    def fire(lt, p):
        for s in range(3):
            for pp in range(2):
                pltpu.async_copy(
                    planes[pp].at[idx_v.at[lt, s]],
                    vb[p].at[s, pp],
                    st[p],
                )

    def wait_tri(p):
        for s in range(3):
            for pp in range(2):
                pltpu.make_async_copy(
                    planes[0].at[idx_v.at[0, 0]],
                    vb[0].at[0, 0],
                    st[p],
                ).wait()

    def wait_out(p):
        for c in range(3):
            pltpu.make_async_copy(
                ob[p].at[c], out_hbm.at[c, 0], so[p]
            ).wait()

    def compute(p):
        for g in range(8):
            rg = 16 * g + lane
            comp = [
                [
                    [
                        plsc.load_gather(
                            vb[p],
                            [jnp.full((16,), s, jnp.int32),
                             jnp.full((16,), b >> 1, jnp.int32),
                             rg,
                             jnp.full((16,), 4 * (b & 1) + c, jnp.int32)],
                        )
                        for c in range(3)
                    ]
                    for b in range(B)
                ]
                for s in range(3)
            ]
            for b in range(B):
                x0, y0, z0 = comp[0][b]
                x1, y1, z1 = comp[1][b]
                x2, y2, z2 = comp[2][b]
                e1x, e1y, e1z = x0 - x1, y0 - y1, z0 - z1
                e2x, e2y, e2z = x2 - x1, y2 - y1, z2 - z1
                nx = e2y * e1z - e2z * e1y
                ny = e2z * e1x - e2x * e1z
                nz = e2x * e1y - e2y * e1x
                sq = nx * nx + ny * ny + nz * nz
                r = _rsqrt16(jnp.maximum(sq, 1e-12))
                for c, val in enumerate((nx * r, ny * r, nz * r)):
                    ob[p][c, b, pl.ds(16 * g, 16)] = val

    def flush(lt, p):
        for c in range(3):
            pltpu.async_copy(ob[p].at[c], out_hbm.at[c, start + lt], so[p])

    for d in range(DEPTH):
        fire(d, d)

    def slot(lt, p):
        wait_tri(p)

        @pl.when(lt >= DEPTH)
        def _():
            wait_out(p)

        compute(p)
        flush(lt, p)

        @pl.when(lt + DEPTH < TPW)
        def _():
            fire(lt + DEPTH, p)

    def outer(o, _):
        for d in range(DEPTH):
            slot(DEPTH * o + d, d)
        return ()

    lax.fori_loop(0, TPW // DEPTH, outer, ())
    wait_tri(0)
    wait_out(0)
    compute(0)
    flush(TPW - 1, 0)
    for d in range(1, DEPTH):
        wait_out(d)
    wait_out(0)


@jax.jit
def kernel(vertices, faces):
    vpad = jnp.pad(vertices, ((0, 0), (0, V_PAD - V), (0, 0)))
    vview = jnp.transpose(
        jnp.transpose(vpad, (2, 1, 0)).reshape(3, NVT, FT, B), (0, 1, 3, 2)
    )

    f32c = faces.astype(jnp.int32)
    fpad = jnp.pad(f32c, ((0, F_PAD - F), (0, 1)))
    fidx = jnp.transpose(fpad.reshape(NT, FT, 4), (0, 2, 1))

    mesh = plsc.VectorSubcoreMesh(core_axis_name="c", subcore_axis_name="s")
    run = pl.kernel(
        _body,
        out_type=jax.ShapeDtypeStruct((3, NT, B, FT), jnp.float32),
        mesh=mesh,
        scratch_types=[pltpu.VMEM_SHARED((V_PAD, 8), jnp.float32)] * 2 + [
            pltpu.VMEM((TPW, 4, FT), jnp.int32),
        ] + [pltpu.VMEM((3, 2, FT, 8), jnp.float32)] * 3 +
        [pltpu.VMEM((3, B, FT), jnp.float32)] * 3 + [
            pltpu.VMEM((12, FT), jnp.float32),
            pltpu.VMEM((12, FT), jnp.float32),
            pltpu.VMEM((2, FT, 8), jnp.float32),
            pltpu.VMEM((2, FT, 8), jnp.float32),
        ] + [pltpu.SemaphoreType.DMA] * 10,
        compiler_params=pltpu.CompilerParams(
            needs_layout_passes=False, use_tc_tiling_on_sc=False
        ),
    )
    out4 = run(vview, fidx)
    y = jnp.transpose(out4, (2, 1, 3, 0)).reshape(B, F_PAD, 3)
    return y[:, :F, :]


# speedup vs baseline: 1.0718x; 1.0718x over previous
import jax
import jax.numpy as jnp
from jax import lax
from jax.experimental import pallas as pl
from jax.experimental.pallas import tpu as pltpu
from jax.experimental.pallas import tpu_sc as plsc

B = 4
V = 50000
F = 100000
FT = 128
NT = (F + FT - 1) // FT
F_PAD = NT * FT
TPW = 25
START_MUL = 25006
NVT = 391
V_PAD = NVT * FT
VTPW = 25
VSTART_MUL = 25008


def _rsqrt16(x):
    i = plsc.bitcast(x, jnp.int32)
    i = jnp.int32(0x5F3759DF) - (i >> 1)
    y = plsc.bitcast(i, jnp.float32)
    xh = 0.5 * x
    for _ in range(2):
        y = y * (1.5 - xh * y * y)
    return y


def _body(verts_hbm, fidx_hbm, out_hbm, *refs):
    planes = refs[0:2]
    idx_v = refs[2]
    vb = refs[3:5]
    ob = refs[5:7]
    cb = refs[7:9]
    rb = refs[9:11]
    st = refs[11:13]
    so = refs[13:15]
    sv = refs[15:17]
    sb = refs[17:19]

    sid = lax.axis_index("s")
    wid = sid * 2 + lax.axis_index("c")
    start = (wid * START_MUL) >> 10
    lane = lax.iota(jnp.int32, 16)

    vstart = (sid * VSTART_MUL) >> 10

    def fire_in(vt, p):
        for c in range(3):
            for b in range(B):
                pltpu.async_copy(
                    verts_hbm.at[c, vt, b], cb[p].at[c * 4 + b], sv[p]
                )

    def wait_in(p):
        for _ in range(12):
            pltpu.make_async_copy(
                verts_hbm.at[0, 0, 0], cb[0].at[0], sv[p]
            ).wait()

    def permute(p):
        for b in range(B):
            for c in range(3):
                col = jnp.full((16,), 4 * (b & 1) + c, jnp.int32)
                for i in range(8):
                    v = cb[p][c * 4 + b, pl.ds(16 * i, 16)]
                    plsc.store_scatter(
                        rb[p], [jnp.full((16,), b >> 1, jnp.int32),
                                16 * i + lane, col], v
                    )

    def fire_bout(vt, p):
        for pp in range(2):
            pltpu.async_copy(
                rb[p].at[pp], planes[pp].at[pl.ds(FT * vt, FT)], sb[p]
            )

    def wait_bout(p):
        for pp in range(2):
            pltpu.make_async_copy(
                rb[0].at[0], planes[0].at[pl.ds(0, FT)], sb[p]
            ).wait()

    fire_in(vstart, 0)

    def bslot(t, p):
        fire_in(t + 1, 1 - p)
        wait_in(p)

        @pl.when(t >= vstart + 2)
        def _():
            wait_bout(p)

        permute(p)
        fire_bout(t, p)

    def bouter(o, _):
        bslot(vstart + 2 * o, 0)
        bslot(vstart + 2 * o + 1, 1)
        return ()

    lax.fori_loop(0, (VTPW - 1) // 2, bouter, ())
    wait_in(0)
    wait_bout(0)
    permute(0)
    fire_bout(vstart + VTPW - 1, 0)
    wait_bout(1)
    wait_bout(0)
    plsc.subcore_barrier()

    pltpu.sync_copy(fidx_hbm.at[pl.ds(start, TPW)], idx_v)

    def fire(lt, p):
        for s in range(3):
            for pp in range(2):
                pltpu.async_copy(
                    planes[pp].at[idx_v.at[lt, s]],
                    vb[p].at[s, pp],
                    st[p],
                )

    def wait_tri(p):
        for s in range(3):
            for pp in range(2):
                pltpu.make_async_copy(
                    planes[0].at[idx_v.at[0, 0]],
                    vb[0].at[0, 0],
                    st[p],
                ).wait()

    def wait_out(p):
        for c in range(3):
            pltpu.make_async_copy(
                ob[p].at[c], out_hbm.at[c, 0], so[p]
            ).wait()

    def compute(p):
        for g in range(8):
            rg = 16 * g + lane
            comp = [
                [
                    [
                        plsc.load_gather(
                            vb[p],
                            [jnp.full((16,), s, jnp.int32),
                             jnp.full((16,), b >> 1, jnp.int32),
                             rg,
                             jnp.full((16,), 4 * (b & 1) + c, jnp.int32)],
                        )
                        for c in range(3)
                    ]
                    for b in range(B)
                ]
                for s in range(3)
            ]
            for b in range(B):
                x0, y0, z0 = comp[0][b]
                x1, y1, z1 = comp[1][b]
                x2, y2, z2 = comp[2][b]
                e1x, e1y, e1z = x0 - x1, y0 - y1, z0 - z1
                e2x, e2y, e2z = x2 - x1, y2 - y1, z2 - z1
                nx = e2y * e1z - e2z * e1y
                ny = e2z * e1x - e2x * e1z
                nz = e2x * e1y - e2y * e1x
                sq = nx * nx + ny * ny + nz * nz
                r = _rsqrt16(jnp.maximum(sq, 1e-12))
                for c, val in enumerate((nx * r, ny * r, nz * r)):
                    ob[p][c, b, pl.ds(16 * g, 16)] = val

    def flush(lt, p):
        for c in range(3):
            pltpu.async_copy(ob[p].at[c], out_hbm.at[c, start + lt], so[p])

    fire(0, 0)

    def slot(lt, p):
        fire(lt + 1, 1 - p)
        wait_tri(p)

        @pl.when(lt >= 2)
        def _():
            wait_out(p)

        compute(p)
        flush(lt, p)

    def outer(o, _):
        slot(2 * o, 0)
        slot(2 * o + 1, 1)
        return ()

    lax.fori_loop(0, (TPW - 1) // 2, outer, ())
    wait_tri(0)
    wait_out(0)
    compute(0)
    flush(TPW - 1, 0)
    wait_out(1)
    wait_out(0)


@jax.jit
def kernel(vertices, faces):
    vpad = jnp.pad(vertices, ((0, 0), (0, V_PAD - V), (0, 0)))
    vview = jnp.transpose(
        jnp.transpose(vpad, (2, 1, 0)).reshape(3, NVT, FT, B), (0, 1, 3, 2)
    )

    f32c = faces.astype(jnp.int32)
    fpad = jnp.pad(f32c, ((0, F_PAD - F), (0, 1)))
    fidx = jnp.transpose(fpad.reshape(NT, FT, 4), (0, 2, 1))

    mesh = plsc.VectorSubcoreMesh(core_axis_name="c", subcore_axis_name="s")
    run = pl.kernel(
        _body,
        out_type=jax.ShapeDtypeStruct((3, NT, B, FT), jnp.float32),
        mesh=mesh,
        scratch_types=[pltpu.VMEM_SHARED((V_PAD, 8), jnp.float32)] * 2 + [
            pltpu.VMEM((TPW, 4, FT), jnp.int32),
        ] + [pltpu.VMEM((3, 2, FT, 8), jnp.float32)] * 2 +
        [pltpu.VMEM((3, B, FT), jnp.float32)] * 2 + [
            pltpu.VMEM((12, FT), jnp.float32),
            pltpu.VMEM((12, FT), jnp.float32),
            pltpu.VMEM((2, FT, 8), jnp.float32),
            pltpu.VMEM((2, FT, 8), jnp.float32),
        ] + [pltpu.SemaphoreType.DMA] * 8,
        compiler_params=pltpu.CompilerParams(
            needs_layout_passes=False, use_tc_tiling_on_sc=False
        ),
    )
    out4 = run(vview, fidx)
    y = jnp.transpose(out4, (2, 1, 3, 0)).reshape(B, F_PAD, 3)
    return y[:, :F, :]
